# baseline (speedup 1.0000x reference)
import functools
import jax
import jax.numpy as jnp
from jax import lax
from jax.experimental import pallas as pl
from jax.experimental.pallas import tpu as pltpu
from jax.experimental.pallas import tpu_sc as plsc

_NT = 576
_NV = 144
_NM = _NT - _NV
_B = 64
_ROWS = _B * _NT
_MC = 48
_NMC = _NM // _MC
_VC = 48
_NVC = _NV // _VC
_NBUF = 2
_NVEC = _NT // 16


def _sc_body(u_hbm, mt_hbm, x_hbm, out_hbm, masks_hbm,
             u_v, masks_v, vis_flat, msk_flat,
             vi0, vi1, vi2, vi3, vi4, vi5,
             mi0, mi1, mi2, mi3, mi4, mi5, mi6, mi7, mi8,
             mi9, mi10, mi11, mi12, mi13, mi14, mi15, mi16, mi17,
             buf0, buf1, mrow,
             semg, semv, semm, semr):
    vis_refs = (vi0, vi1, vi2, vi3, vi4, vi5)
    msk_refs = (mi0, mi1, mi2, mi3, mi4, mi5, mi6, mi7, mi8,
                mi9, mi10, mi11, mi12, mi13, mi14, mi15, mi16, mi17)
    bufs = (buf0, buf1)
    wid = lax.axis_index("s") * 2 + lax.axis_index("c")
    row0 = wid * 2 * _NT

    hmrow = pltpu.async_copy(mt_hbm, mrow, semr)

    lane = lax.iota(jnp.int32, 16)

    pltpu.sync_copy(u_hbm.at[pl.ds(row0, 2 * _NT)], u_v)

    def compute_batch(par):
        off = par * _NT

        def cnt_le(mid):
            acc = jnp.zeros((16,), jnp.int32)
            midv = jnp.full((16,), mid, jnp.int32)
            for i in range(_NVEC):
                uv = u_v[pl.ds(off + i * 16, 16)]
                acc = acc + (uv <= midv).astype(jnp.int32)
            return jnp.sum(acc)

        @pl.loop(0, 30, init_carry=(jnp.int32(0), jnp.int32(0x3F800000)))
        def bisect(_, c):
            lo, hi = c
            mid = lax.div(lo + hi, 2)
            big = cnt_le(mid) >= _NV
            return (jnp.where(big, lo, mid + 1), jnp.where(big, mid, hi))

        lo, _ = bisect
        vstar = jnp.full((16,), lo, jnp.int32)

        acc = jnp.zeros((16,), jnp.int32)
        for i in range(_NVEC):
            uv = u_v[pl.ds(off + i * 16, 16)]
            acc = acc + (uv < vstar).astype(jnp.int32)
        rem = _NV - jnp.sum(acc)
        remv = jnp.full((16,), rem, jnp.int32)

        ecarry = jnp.int32(0)
        vcarry = jnp.int32(0)
        mcarry = jnp.int32(0)
        for i in range(_NVEC):
            uv = u_v[pl.ds(off + i * 16, 16)]
            meq = uv == vstar
            e = meq.astype(jnp.int32)
            ce = lax.cumsum(e)
            tie_idx = jnp.full((16,), ecarry, jnp.int32) + ce - e
            vis = (uv < vstar) | (meq & (tie_idx < remv))
            ecarry = ecarry + jnp.sum(e)
            v = vis.astype(jnp.int32)
            masks_v[pl.ds(off + i * 16, 16)] = 1.0 - vis.astype(jnp.float32)
            ids = jnp.full((16,), row0 + off + i * 16, jnp.int32) + lane
            cv = lax.cumsum(v)
            pos_v = jnp.full((16,), vcarry, jnp.int32) + cv - v
            plsc.store_scatter(vis_flat, [pos_v], ids, mask=vis)
            vcarry = vcarry + jnp.sum(v)
            m = 1 - v
            cm = lax.cumsum(m)
            pos_m = jnp.full((16,), mcarry, jnp.int32) + cm - m
            plsc.store_scatter(msk_flat, [pos_m], ids, mask=~vis)
            mcarry = mcarry + jnp.sum(m)

        for c in range(_NVC):
            g = par * _NVC + c
            for k in range(_VC // 16):
                vis_refs[g][pl.ds(k * 16, 16)] = vis_flat[pl.ds(c * _VC + k * 16, 16)]
        for c in range(_NMC):
            g = par * _NMC + c
            for k in range(_MC // 16):
                msk_refs[g][pl.ds(k * 16, 16)] = msk_flat[pl.ds(c * _MC + k * 16, 16)]

    def gather(c):
        return pltpu.async_copy(x_hbm.at[vis_refs[c]], bufs[c % _NBUF], semg)

    def fire_msk(par):
        return [pltpu.async_copy(mrow, out_hbm.at[msk_refs[par * _NMC + c]], semm)
                for c in range(_NMC)]

    compute_batch(0)
    hmrow.wait()
    hm0 = fire_msk(0)
    hg = {c: gather(c) for c in range(_NBUF)}
    compute_batch(1)
    hm1 = fire_msk(1)
    hr = pltpu.async_copy(masks_v, masks_hbm.at[pl.ds(row0, 2 * _NT)], semr)

    hv = {}
    ntot = 2 * _NVC
    for c in range(ntot):
        hg[c].wait()
        hv[c] = pltpu.async_copy(bufs[c % _NBUF], out_hbm.at[vis_refs[c]], semv)
        if c + 1 < ntot and c + 1 >= _NBUF:
            hv[c + 1 - _NBUF].wait()
            hg[c + 1] = gather(c + 1)
    for c in range(ntot - _NBUF, ntot):
        hv[c].wait()
    for h in hm0:
        h.wait()
    for h in hm1:
        h.wait()
    hr.wait()


def kernel(x, mask_token):
    b, d = x.shape[0], x.shape[-1]
    noise = jax.random.uniform(jax.random.key(42), (b, 1, _NT), dtype=jnp.float32)
    u = lax.bitcast_convert_type(noise.reshape(b * _NT), jnp.int32)
    x_rows = x.reshape(_ROWS, d)

    mesh = plsc.VectorSubcoreMesh(core_axis_name="c", subcore_axis_name="s")
    fn = functools.partial(
        pl.kernel,
        mesh=mesh,
        compiler_params=pltpu.CompilerParams(needs_layout_passes=False),
        out_type=[
            jax.ShapeDtypeStruct((_ROWS, d), jnp.float32),
            jax.ShapeDtypeStruct((_ROWS,), jnp.float32),
        ],
        scratch_types=[
            pltpu.VMEM((2 * _NT,), jnp.int32),
            pltpu.VMEM((2 * _NT,), jnp.float32),
            pltpu.VMEM((_NV + 16,), jnp.int32),
            pltpu.VMEM((_NM + 16,), jnp.int32),
        ] + [pltpu.VMEM((_VC,), jnp.int32) for _ in range(2 * _NVC)]
        + [pltpu.VMEM((_MC,), jnp.int32) for _ in range(2 * _NMC)]
        + [pltpu.VMEM((_VC, d), jnp.float32) for _ in range(_NBUF)]
        + [
            pltpu.VMEM((_MC, d), jnp.float32),
            pltpu.SemaphoreType.DMA,
            pltpu.SemaphoreType.DMA,
            pltpu.SemaphoreType.DMA,
            pltpu.SemaphoreType.DMA,
        ],
    )(_sc_body)
    mt_rep = jnp.broadcast_to(mask_token[None, :], (_MC, d))
    out, masks = fn(u, mt_rep, x_rows)
    return out.reshape(x.shape), masks.reshape(b, _NT)


# speedup vs baseline: 1.0527x; 1.0527x over previous
import functools
import jax
import jax.numpy as jnp
from jax import lax
from jax.experimental import pallas as pl
from jax.experimental.pallas import tpu as pltpu
from jax.experimental.pallas import tpu_sc as plsc

_NT = 576
_NV = 144
_NM = _NT - _NV
_B = 64
_ROWS = _B * _NT
_MC = 48
_NMC = _NM // _MC
_VC = 48
_NVC = _NV // _VC
_NBUF = 2
_NVEC = _NT // 16


def _sc_body(u_hbm, mt_hbm, x_hbm, out_hbm, masks_hbm,
             u_v, masks_v, vis_flat, msk_flat,
             vi0, vi1, vi2, vi3, vi4, vi5,
             mi0, mi1, mi2, mi3, mi4, mi5, mi6, mi7, mi8,
             mi9, mi10, mi11, mi12, mi13, mi14, mi15, mi16, mi17,
             buf0, buf1, mrow,
             semg, semv, semm, semr):
    vis_refs = (vi0, vi1, vi2, vi3, vi4, vi5)
    msk_refs = (mi0, mi1, mi2, mi3, mi4, mi5, mi6, mi7, mi8,
                mi9, mi10, mi11, mi12, mi13, mi14, mi15, mi16, mi17)
    bufs = (buf0, buf1)
    wid = lax.axis_index("s") * 2 + lax.axis_index("c")
    row0 = wid * 2 * _NT

    pltpu.sync_copy(mt_hbm, mrow)

    lane = lax.iota(jnp.int32, 16)

    pltpu.sync_copy(u_hbm.at[pl.ds(row0, 2 * _NT)], u_v)

    def compute_batch(par):
        off = par * _NT

        def cnt_le(mid):
            acc = jnp.zeros((16,), jnp.int32)
            midv = jnp.full((16,), mid, jnp.int32)
            for i in range(_NVEC):
                uv = u_v[pl.ds(off + i * 16, 16)]
                acc = acc + (uv <= midv).astype(jnp.int32)
            return jnp.sum(acc)

        @pl.loop(0, 30, init_carry=(jnp.int32(0), jnp.int32(0x3F800000)))
        def bisect(_, c):
            lo, hi = c
            mid = lax.div(lo + hi, 2)
            big = cnt_le(mid) >= _NV
            return (jnp.where(big, lo, mid + 1), jnp.where(big, mid, hi))

        lo, _ = bisect
        vstar = jnp.full((16,), lo, jnp.int32)

        acc = jnp.zeros((16,), jnp.int32)
        for i in range(_NVEC):
            uv = u_v[pl.ds(off + i * 16, 16)]
            acc = acc + (uv < vstar).astype(jnp.int32)
        rem = _NV - jnp.sum(acc)
        remv = jnp.full((16,), rem, jnp.int32)

        ecarry = jnp.int32(0)
        vcarry = jnp.int32(0)
        mcarry = jnp.int32(0)
        for i in range(_NVEC):
            uv = u_v[pl.ds(off + i * 16, 16)]
            meq = uv == vstar
            e = meq.astype(jnp.int32)
            ce = lax.cumsum(e)
            tie_idx = jnp.full((16,), ecarry, jnp.int32) + ce - e
            vis = (uv < vstar) | (meq & (tie_idx < remv))
            ecarry = ecarry + jnp.sum(e)
            v = vis.astype(jnp.int32)
            masks_v[pl.ds(off + i * 16, 16)] = 1.0 - vis.astype(jnp.float32)
            ids = jnp.full((16,), row0 + off + i * 16, jnp.int32) + lane
            cv = lax.cumsum(v)
            pos_v = jnp.full((16,), vcarry, jnp.int32) + cv - v
            plsc.store_scatter(vis_flat, [pos_v], ids, mask=vis)
            vcarry = vcarry + jnp.sum(v)
            m = 1 - v
            cm = lax.cumsum(m)
            pos_m = jnp.full((16,), mcarry, jnp.int32) + cm - m
            plsc.store_scatter(msk_flat, [pos_m], ids, mask=~vis)
            mcarry = mcarry + jnp.sum(m)

        for c in range(_NVC):
            g = par * _NVC + c
            for k in range(_VC // 16):
                vis_refs[g][pl.ds(k * 16, 16)] = vis_flat[pl.ds(c * _VC + k * 16, 16)]
        for c in range(_NMC):
            g = par * _NMC + c
            for k in range(_MC // 16):
                msk_refs[g][pl.ds(k * 16, 16)] = msk_flat[pl.ds(c * _MC + k * 16, 16)]

    def gather(c):
        return pltpu.async_copy(x_hbm.at[vis_refs[c]], bufs[c % _NBUF], semg)

    def fire_msk(par):
        return [pltpu.async_copy(mrow, out_hbm.at[msk_refs[par * _NMC + c]], semm)
                for c in range(_NMC)]

    compute_batch(0)
    hm0 = fire_msk(0)
    hg = {c: gather(c) for c in range(_NBUF)}
    compute_batch(1)
    hm1 = fire_msk(1)
    hr = pltpu.async_copy(masks_v, masks_hbm.at[pl.ds(row0, 2 * _NT)], semr)

    hv = {}
    ntot = 2 * _NVC
    for c in range(ntot):
        hg[c].wait()
        hv[c] = pltpu.async_copy(bufs[c % _NBUF], out_hbm.at[vis_refs[c]], semv)
        if c + 1 < ntot and c + 1 >= _NBUF:
            hv[c + 1 - _NBUF].wait()
            hg[c + 1] = gather(c + 1)
    for c in range(ntot - _NBUF, ntot):
        hv[c].wait()
    for h in hm0:
        h.wait()
    for h in hm1:
        h.wait()
    hr.wait()


def kernel(x, mask_token):
    b, d = x.shape[0], x.shape[-1]
    noise = jax.random.uniform(jax.random.key(42), (b, 1, _NT), dtype=jnp.float32)
    u = lax.bitcast_convert_type(noise.reshape(b * _NT), jnp.int32)
    x_rows = x.reshape(_ROWS, d)

    mesh = plsc.VectorSubcoreMesh(core_axis_name="c", subcore_axis_name="s")
    fn = functools.partial(
        pl.kernel,
        mesh=mesh,
        compiler_params=pltpu.CompilerParams(needs_layout_passes=False),
        out_type=[
            jax.ShapeDtypeStruct((_ROWS, d), jnp.float32),
            jax.ShapeDtypeStruct((_ROWS,), jnp.float32),
        ],
        scratch_types=[
            pltpu.VMEM((2 * _NT,), jnp.int32),
            pltpu.VMEM((2 * _NT,), jnp.float32),
            pltpu.VMEM((_NV + 16,), jnp.int32),
            pltpu.VMEM((_NM + 16,), jnp.int32),
        ] + [pltpu.VMEM((_VC,), jnp.int32) for _ in range(2 * _NVC)]
        + [pltpu.VMEM((_MC,), jnp.int32) for _ in range(2 * _NMC)]
        + [pltpu.VMEM((_VC, d), jnp.float32) for _ in range(_NBUF)]
        + [
            pltpu.VMEM((_MC, d), jnp.float32),
            pltpu.SemaphoreType.DMA,
            pltpu.SemaphoreType.DMA,
            pltpu.SemaphoreType.DMA,
            pltpu.SemaphoreType.DMA,
        ],
    )(_sc_body)
    mt_rep = jnp.broadcast_to(mask_token[None, :], (_MC, d))
    out, masks = fn(u, mt_rep, x_rows)
    return out.reshape(x.shape), masks.reshape(b, _NT)
